# baseline (speedup 1.0000x reference)
import functools
import math

import jax
import jax.numpy as jnp
from jax import lax
from jax.experimental import pallas as pl
from jax.experimental.pallas import tpu as pltpu
from jax.experimental.pallas import tpu_sc as plsc

D_MODEL = 64
SCALE = math.sqrt(D_MODEL)
NUM_CORES = 2
NUM_SUBCORES = 16
NUM_WORKERS = NUM_CORES * NUM_SUBCORES
CHUNK = 512
LANES = 16


@functools.lru_cache(maxsize=None)
def _make_emb_kernel(B: int, V: int):
    assert B % (NUM_WORKERS * CHUNK) == 0
    b_per_w = B // NUM_WORKERS
    n_chunks = b_per_w // CHUNK
    assert n_chunks % 2 == 0
    mesh = plsc.VectorSubcoreMesh(
        core_axis_name="c",
        subcore_axis_name="s",
        num_cores=NUM_CORES,
        num_subcores=NUM_SUBCORES,
    )

    @functools.partial(
        pl.kernel,
        out_type=jax.ShapeDtypeStruct((B, D_MODEL), jnp.float32),
        mesh=mesh,
        scratch_types=[
            pltpu.VMEM((CHUNK,), jnp.int32),
            pltpu.VMEM((CHUNK,), jnp.int32),
            pltpu.VMEM((CHUNK, D_MODEL), jnp.float32),
            pltpu.VMEM((CHUNK, D_MODEL), jnp.float32),
            pltpu.SemaphoreType.DMA,
            pltpu.SemaphoreType.DMA,
            pltpu.SemaphoreType.DMA,
            pltpu.SemaphoreType.DMA,
        ],
        compiler_params=pltpu.CompilerParams(use_tc_tiling_on_sc=False),
    )
    def emb(x_hbm, table_hbm, out_hbm, i0, i1, r0, r1,
            gsem0, gsem1, osem0, osem1):
        wid = lax.axis_index("s") * NUM_CORES + lax.axis_index("c")
        base = wid * b_per_w
        ibufs, rbufs = (i0, i1), (r0, r1)
        gsems, osems = (gsem0, gsem1), (osem0, osem1)

        def scopy(k, b):
            return pltpu.make_async_copy(
                rbufs[b], out_hbm.at[pl.ds(base + k * CHUNK, CHUNK)],
                osems[b])

        def stage_and_gather(k, b):
            pltpu.sync_copy(x_hbm.at[pl.ds(base + k * CHUNK, CHUNK)],
                            ibufs[b])
            pltpu.async_copy(table_hbm.at[ibufs[b]], rbufs[b], gsems[b])

        def gwait(b):
            pltpu.make_async_copy(table_hbm.at[ibufs[b]], rbufs[b],
                                  gsems[b]).wait()

        def scale(b):
            def row_body(r, c):
                for j in range(D_MODEL // LANES):
                    sl = pl.ds(j * LANES, LANES)
                    rbufs[b][r, sl] = rbufs[b][r, sl] * SCALE
                return c

            lax.fori_loop(0, CHUNK, row_body, 0, unroll=4)

        stage_and_gather(0, 0)

        def pair_body(h, carry):
            for b in range(2):
                k = 2 * h + b
                b1 = 1 - b
                gwait(b)
                scale(b)
                scopy(k, b).start()

                @pl.when((k >= 1) & (k + 1 < n_chunks))
                def _():
                    scopy(k - 1, b1).wait()

                @pl.when(k + 1 < n_chunks)
                def _():
                    stage_and_gather(k + 1, b1)
            return carry

        lax.fori_loop(0, n_chunks // 2, pair_body, 0)
        scopy(n_chunks - 2, 0).wait()
        scopy(n_chunks - 1, 1).wait()

    return emb


def kernel(x, table):
    B = x.size
    xf = x.reshape(B).astype(jnp.int32)
    out = _make_emb_kernel(B, table.shape[0])(xf, table)
    return out.reshape(*x.shape, D_MODEL)


# speedup vs baseline: 1.0017x; 1.0017x over previous
import functools
import math

import jax
import jax.numpy as jnp
from jax import lax
from jax.experimental import pallas as pl
from jax.experimental.pallas import tpu as pltpu
from jax.experimental.pallas import tpu_sc as plsc

D_MODEL = 64
SCALE = math.sqrt(D_MODEL)
NUM_CORES = 2
NUM_SUBCORES = 16
NUM_WORKERS = NUM_CORES * NUM_SUBCORES
CHUNK = 512
LANES = 16


@functools.lru_cache(maxsize=None)
def _make_emb_kernel(B: int, V: int):
    assert B % (NUM_WORKERS * CHUNK) == 0
    b_per_w = B // NUM_WORKERS
    n_chunks = b_per_w // CHUNK
    assert n_chunks % 2 == 0
    mesh = plsc.VectorSubcoreMesh(
        core_axis_name="c",
        subcore_axis_name="s",
        num_cores=NUM_CORES,
        num_subcores=NUM_SUBCORES,
    )

    @functools.partial(
        pl.kernel,
        out_type=jax.ShapeDtypeStruct((B, D_MODEL), jnp.float32),
        mesh=mesh,
        scratch_types=[
            pltpu.VMEM((CHUNK,), jnp.int32),
            pltpu.VMEM((CHUNK,), jnp.int32),
            pltpu.VMEM((CHUNK, D_MODEL), jnp.float32),
            pltpu.VMEM((CHUNK, D_MODEL), jnp.float32),
            pltpu.SemaphoreType.DMA,
            pltpu.SemaphoreType.DMA,
            pltpu.SemaphoreType.DMA,
            pltpu.SemaphoreType.DMA,
        ],
        compiler_params=pltpu.CompilerParams(use_tc_tiling_on_sc=False),
    )
    def emb(x_hbm, table_hbm, out_hbm, i0, i1, r0, r1,
            gsem0, gsem1, osem0, osem1):
        wid = lax.axis_index("s") * NUM_CORES + lax.axis_index("c")
        base = wid * b_per_w
        ibufs, rbufs = (i0, i1), (r0, r1)
        gsems, osems = (gsem0, gsem1), (osem0, osem1)

        def scopy(k, b):
            return pltpu.make_async_copy(
                rbufs[b], out_hbm.at[pl.ds(base + k * CHUNK, CHUNK)],
                osems[b])

        def chunk_work(k, b):
            off = base + k * CHUNK
            pltpu.sync_copy(x_hbm.at[pl.ds(off, CHUNK)], ibufs[b])
            pltpu.async_copy(table_hbm.at[ibufs[b]], rbufs[b],
                             gsems[b]).wait()

            def row_body(r, c):
                for j in range(D_MODEL // LANES):
                    sl = pl.ds(j * LANES, LANES)
                    rbufs[b][r, sl] = rbufs[b][r, sl] * SCALE
                return c

            lax.fori_loop(0, CHUNK, row_body, 0, unroll=4)
            scopy(k, b).start()

        def pair_body(h, carry):
            for b in range(2):
                k = 2 * h + b

                @pl.when(k >= 2)
                def _():
                    scopy(k - 2, b).wait()

                chunk_work(k, b)
            return carry

        lax.fori_loop(0, n_chunks // 2, pair_body, 0)
        for b in range(2):
            scopy(n_chunks - 2 + b, b).wait()

    return emb


def kernel(x, table):
    B = x.size
    xf = x.reshape(B).astype(jnp.int32)
    out = _make_emb_kernel(B, table.shape[0])(xf, table)
    return out.reshape(*x.shape, D_MODEL)


# speedup vs baseline: 1.0096x; 1.0079x over previous
import functools
import math

import jax
import jax.numpy as jnp
from jax import lax
from jax.experimental import pallas as pl
from jax.experimental.pallas import tpu as pltpu
from jax.experimental.pallas import tpu_sc as plsc

D_MODEL = 64
SCALE = math.sqrt(D_MODEL)
NUM_CORES = 2
NUM_SUBCORES = 16
NUM_WORKERS = NUM_CORES * NUM_SUBCORES
CHUNK = 640
LANES = 16


@functools.lru_cache(maxsize=None)
def _make_emb_kernel(B: int, V: int):
    assert B % (NUM_WORKERS * CHUNK) == 0
    b_per_w = B // NUM_WORKERS
    n_chunks = b_per_w // CHUNK
    assert n_chunks % 2 == 0
    mesh = plsc.VectorSubcoreMesh(
        core_axis_name="c",
        subcore_axis_name="s",
        num_cores=NUM_CORES,
        num_subcores=NUM_SUBCORES,
    )

    @functools.partial(
        pl.kernel,
        out_type=jax.ShapeDtypeStruct((B, D_MODEL), jnp.float32),
        mesh=mesh,
        scratch_types=[
            pltpu.VMEM((CHUNK,), jnp.int32),
            pltpu.VMEM((CHUNK,), jnp.int32),
            pltpu.VMEM((CHUNK, D_MODEL), jnp.float32),
            pltpu.VMEM((CHUNK, D_MODEL), jnp.float32),
            pltpu.SemaphoreType.DMA,
            pltpu.SemaphoreType.DMA,
            pltpu.SemaphoreType.DMA,
            pltpu.SemaphoreType.DMA,
        ],
        compiler_params=pltpu.CompilerParams(use_tc_tiling_on_sc=False),
    )
    def emb(x_hbm, table_hbm, out_hbm, i0, i1, r0, r1,
            gsem0, gsem1, osem0, osem1):
        wid = lax.axis_index("s") * NUM_CORES + lax.axis_index("c")
        base = wid * b_per_w
        ibufs, rbufs = (i0, i1), (r0, r1)
        gsems, osems = (gsem0, gsem1), (osem0, osem1)

        def scopy(k, b):
            return pltpu.make_async_copy(
                rbufs[b], out_hbm.at[pl.ds(base + k * CHUNK, CHUNK)],
                osems[b])

        def chunk_work(k, b):
            off = base + k * CHUNK
            pltpu.sync_copy(x_hbm.at[pl.ds(off, CHUNK)], ibufs[b])
            pltpu.async_copy(table_hbm.at[ibufs[b]], rbufs[b],
                             gsems[b]).wait()

            def row_body(r, c):
                for j in range(D_MODEL // LANES):
                    sl = pl.ds(j * LANES, LANES)
                    rbufs[b][r, sl] = rbufs[b][r, sl] * SCALE
                return c

            lax.fori_loop(0, CHUNK, row_body, 0, unroll=4)
            scopy(k, b).start()

        def pair_body(h, carry):
            for b in range(2):
                k = 2 * h + b

                @pl.when(k >= 2)
                def _():
                    scopy(k - 2, b).wait()

                chunk_work(k, b)
            return carry

        lax.fori_loop(0, n_chunks // 2, pair_body, 0)
        for b in range(2):
            scopy(n_chunks - 2 + b, b).wait()

    return emb


def kernel(x, table):
    B = x.size
    xf = x.reshape(B).astype(jnp.int32)
    out = _make_emb_kernel(B, table.shape[0])(xf, table)
    return out.reshape(*x.shape, D_MODEL)
